# Initial kernel scaffold; baseline (speedup 1.0000x reference)
import functools

import jax
import jax.numpy as jnp
from jax import lax
from jax.experimental import pallas as pl
from jax.experimental.pallas import tpu as pltpu
from jax.experimental.pallas import tpu_sc as plsc

N = 10000
E = 320000
IN_DIM = 128
HID = 16
HEADS = 8
OUT_DIM = 16

NC, NS = 2, 16
NW = NC * NS
EB = 125
CPW = E // (NW * EB)
ROWS = E // EB
RPT = N // NS

NEG = 0.2
_DOT = dict(preferred_element_type=jnp.float32, precision=lax.Precision.HIGHEST)


def _prep1_body(x_ref, w1_ref, bs_ref, bd_ref, h_ref, sp_ref, dp_ref):
    h = jnp.dot(x_ref[...], w1_ref[...], **_DOT)
    s = jnp.dot(h, bs_ref[...], **_DOT)
    d = jnp.dot(h, bd_ref[...], **_DOT)
    z = s + d
    c = jnp.maximum(z, NEG * z)
    lane = lax.broadcasted_iota(jnp.int32, c.shape, 1)
    eneg = jnp.where(lane < HEADS, jnp.exp(-c), 0.0)
    h_ref[...] = h
    sp_ref[...] = s
    dp_ref[...] = jnp.concatenate([d, eneg], axis=1)


def _prep1(x, W1, Bs, Bd, blk=1000):
    grid = (N // blk,)
    return pl.pallas_call(
        _prep1_body,
        grid=grid,
        in_specs=[
            pl.BlockSpec((blk, IN_DIM), lambda i: (i, 0)),
            pl.BlockSpec((IN_DIM, IN_DIM), lambda i: (0, 0)),
            pl.BlockSpec((IN_DIM, 16), lambda i: (0, 0)),
            pl.BlockSpec((IN_DIM, 16), lambda i: (0, 0)),
        ],
        out_specs=[
            pl.BlockSpec((blk, IN_DIM), lambda i: (i, 0)),
            pl.BlockSpec((blk, 16), lambda i: (i, 0)),
            pl.BlockSpec((blk, 32), lambda i: (i, 0)),
        ],
        out_shape=[
            jax.ShapeDtypeStruct((N, IN_DIM), jnp.float32),
            jax.ShapeDtypeStruct((N, 16), jnp.float32),
            jax.ShapeDtypeStruct((N, 32), jnp.float32),
        ],
    )(x, W1, Bs, Bd)


def _edge_phase_a(spack, dpack, src_rows, dst_rows):
    mesh = plsc.VectorSubcoreMesh(
        core_axis_name="c", subcore_axis_name="s", num_cores=NC, num_subcores=NS
    )

    @functools.partial(
        pl.kernel,
        out_type=[
            jax.ShapeDtypeStruct((NC, N, 16), jnp.float32),
            jax.ShapeDtypeStruct((E, 16), jnp.float32),
        ],
        mesh=mesh,
        scratch_types=[
            pltpu.VMEM((CPW, EB), jnp.int32),
            pltpu.VMEM((CPW, EB), jnp.int32),
            pltpu.VMEM((EB, 16), jnp.float32),
            pltpu.VMEM((EB, 32), jnp.float32),
            pltpu.VMEM((EB, 16), jnp.float32),
            pltpu.VMEM_SHARED((N, 16), jnp.float32),
        ],
    )
    def k(sp_hbm, dp_hbm, src_hbm, dst_hbm, den_hbm, ea_hbm,
          ids_s, ids_d, sbuf, dbuf, ebuf, den_sh):
        ci = lax.axis_index("c")
        si = lax.axis_index("s")
        w = si * NC + ci
        base_row = w * CPW
        pltpu.sync_copy(src_hbm.at[pl.ds(base_row, CPW)], ids_s)
        pltpu.sync_copy(dst_hbm.at[pl.ds(base_row, CPW)], ids_d)

        @pl.loop(0, EB)
        def _(i):
            ebuf[i, :] = jnp.zeros((16,), jnp.float32)

        @pl.loop(0, RPT // EB)
        def _(r):
            pltpu.sync_copy(ebuf, den_sh.at[pl.ds(si * RPT + r * EB, EB)])

        plsc.subcore_barrier()

        @pl.loop(0, CPW)
        def _(j):
            pltpu.sync_copy(sp_hbm.at[ids_s.at[j]], sbuf)
            pltpu.sync_copy(dp_hbm.at[ids_d.at[j]], dbuf)

            @pl.loop(0, EB)
            def _(e):
                sv = sbuf[e, :]
                d1 = dbuf[e, pl.ds(0, 16)]
                d2 = dbuf[e, pl.ds(16, 16)]
                z = sv + d1
                lz = jnp.maximum(z, NEG * z)
                ebuf[e, :] = jnp.exp(lz) * d2

            pltpu.sync_copy(ebuf, den_sh.at[ids_d.at[j]], add=True)
            pltpu.sync_copy(ebuf, ea_hbm.at[pl.ds((base_row + j) * EB, EB)])

        plsc.subcore_barrier()
        pltpu.sync_copy(den_sh.at[pl.ds(si * RPT, RPT)],
                        den_hbm.at[ci].at[pl.ds(si * RPT, RPT)])

    return k(spack, dpack, src_rows, dst_rows)


def _edge_phase_b(h, ealpha, src_rows, dst_rows, D):
    nvec = D // 16
    mesh = plsc.VectorSubcoreMesh(
        core_axis_name="c", subcore_axis_name="s", num_cores=NC, num_subcores=NS
    )

    @functools.partial(
        pl.kernel,
        out_type=jax.ShapeDtypeStruct((NC, N, D), jnp.float32),
        mesh=mesh,
        scratch_types=[
            pltpu.VMEM((CPW, EB), jnp.int32),
            pltpu.VMEM((CPW, EB), jnp.int32),
            pltpu.VMEM((EB, D), jnp.float32),
            pltpu.VMEM((EB, 16), jnp.float32),
            pltpu.VMEM_SHARED((N, D), jnp.float32),
        ],
    )
    def k(h_hbm, ea_hbm, src_hbm, dst_hbm, out_hbm,
          ids_s, ids_d, hbuf, ebuf, out_sh):
        ci = lax.axis_index("c")
        si = lax.axis_index("s")
        w = si * NC + ci
        base_row = w * CPW
        pltpu.sync_copy(src_hbm.at[pl.ds(base_row, CPW)], ids_s)
        pltpu.sync_copy(dst_hbm.at[pl.ds(base_row, CPW)], ids_d)

        @pl.loop(0, EB)
        def _(i):
            for v in range(nvec):
                hbuf[i, pl.ds(v * 16, 16)] = jnp.zeros((16,), jnp.float32)

        @pl.loop(0, RPT // EB)
        def _(r):
            pltpu.sync_copy(hbuf, out_sh.at[pl.ds(si * RPT + r * EB, EB)])

        plsc.subcore_barrier()

        @pl.loop(0, CPW)
        def _(j):
            pltpu.sync_copy(h_hbm.at[ids_s.at[j]], hbuf)
            pltpu.sync_copy(ea_hbm.at[pl.ds((base_row + j) * EB, EB)], ebuf)

            @pl.loop(0, EB)
            def _(e):
                for v in range(nvec):
                    ce = ebuf[e, v]
                    hv = hbuf[e, pl.ds(v * 16, 16)]
                    hbuf[e, pl.ds(v * 16, 16)] = hv * ce

            pltpu.sync_copy(hbuf, out_sh.at[ids_d.at[j]], add=True)

        plsc.subcore_barrier()
        pltpu.sync_copy(out_sh.at[pl.ds(si * RPT, RPT)],
                        out_hbm.at[ci].at[pl.ds(si * RPT, RPT)])

    return k(h, ealpha, src_rows, dst_rows)


def _comb1_body(p0_ref, p1_ref, d0_ref, d1_ref, h1_ref, w2_ref, b2s_ref,
                b2d_ref, b1_ref, rx_ref, h2_ref, sp2_ref, dp2_ref):
    den = d0_ref[...] + d1_ref[...] + 1.0
    rcp = 1.0 / (den + 1e-16)
    rexp = jnp.dot(rcp, rx_ref[...], **_DOT)
    g = (p0_ref[...] + p1_ref[...] + h1_ref[...]) * rexp + b1_ref[...]
    g = jnp.where(g > 0, g, jnp.expm1(g))
    h2 = jnp.dot(g, w2_ref[...], **_DOT)
    s2 = jnp.dot(h2, b2s_ref[...], **_DOT)
    d2 = jnp.dot(h2, b2d_ref[...], **_DOT)
    z = s2 + d2
    c2 = jnp.maximum(z, NEG * z)
    lane = lax.broadcasted_iota(jnp.int32, c2.shape, 1)
    eneg = jnp.where(lane < 1, jnp.exp(-c2), 0.0)
    h2_ref[...] = h2
    sp2_ref[...] = s2
    dp2_ref[...] = jnp.concatenate([d2, eneg], axis=1)


def _comb1(p0, p1, d0, d1, h1, W2, B2s, B2d, b1, rxmat, blk=1000):
    grid = (N // blk,)
    return pl.pallas_call(
        _comb1_body,
        grid=grid,
        in_specs=[
            pl.BlockSpec((blk, IN_DIM), lambda i: (i, 0)),
            pl.BlockSpec((blk, IN_DIM), lambda i: (i, 0)),
            pl.BlockSpec((blk, 16), lambda i: (i, 0)),
            pl.BlockSpec((blk, 16), lambda i: (i, 0)),
            pl.BlockSpec((blk, IN_DIM), lambda i: (i, 0)),
            pl.BlockSpec((IN_DIM, OUT_DIM), lambda i: (0, 0)),
            pl.BlockSpec((OUT_DIM, 16), lambda i: (0, 0)),
            pl.BlockSpec((OUT_DIM, 16), lambda i: (0, 0)),
            pl.BlockSpec((1, IN_DIM), lambda i: (0, 0)),
            pl.BlockSpec((16, IN_DIM), lambda i: (0, 0)),
        ],
        out_specs=[
            pl.BlockSpec((blk, OUT_DIM), lambda i: (i, 0)),
            pl.BlockSpec((blk, 16), lambda i: (i, 0)),
            pl.BlockSpec((blk, 32), lambda i: (i, 0)),
        ],
        out_shape=[
            jax.ShapeDtypeStruct((N, OUT_DIM), jnp.float32),
            jax.ShapeDtypeStruct((N, 16), jnp.float32),
            jax.ShapeDtypeStruct((N, 32), jnp.float32),
        ],
    )(p0, p1, d0, d1, h1, W2, B2s, B2d, b1, rxmat)


def _comb2_body(q0_ref, q1_ref, dd0_ref, dd1_ref, h2_ref, b2_ref, cx_ref,
                out_ref):
    den = dd0_ref[...] + dd1_ref[...] + 1.0
    rcp = 1.0 / (den + 1e-16)
    r0 = jnp.dot(rcp, cx_ref[...], **_DOT)
    out_ref[...] = (q0_ref[...] + q1_ref[...] + h2_ref[...]) * r0 + b2_ref[...]


def _comb2(q0, q1, dd0, dd1, h2, b2, cxmat, blk=1000):
    grid = (N // blk,)
    return pl.pallas_call(
        _comb2_body,
        grid=grid,
        in_specs=[
            pl.BlockSpec((blk, OUT_DIM), lambda i: (i, 0)),
            pl.BlockSpec((blk, OUT_DIM), lambda i: (i, 0)),
            pl.BlockSpec((blk, 16), lambda i: (i, 0)),
            pl.BlockSpec((blk, 16), lambda i: (i, 0)),
            pl.BlockSpec((blk, OUT_DIM), lambda i: (i, 0)),
            pl.BlockSpec((1, OUT_DIM), lambda i: (0, 0)),
            pl.BlockSpec((16, OUT_DIM), lambda i: (0, 0)),
        ],
        out_specs=pl.BlockSpec((blk, OUT_DIM), lambda i: (i, 0)),
        out_shape=jax.ShapeDtypeStruct((N, OUT_DIM), jnp.float32),
    )(q0, q1, dd0, dd1, h2, b2, cxmat)


def _head_proj(att, heads, hid):
    m = jnp.zeros((heads * hid, 16), jnp.float32)
    rows = jnp.arange(heads * hid)
    cols = jnp.repeat(jnp.arange(heads), hid)
    return m.at[rows, cols].set(att.reshape(-1))


def kernel(x, edge_index, W1, att_src1, att_dst1, b1, W2, att_src2, att_dst2,
           b2):
    src_rows = edge_index[0].reshape(ROWS, EB)
    dst_rows = edge_index[1].reshape(ROWS, EB)

    Bs1 = _head_proj(att_src1, HEADS, HID)
    Bd1 = _head_proj(att_dst1, HEADS, HID)
    B2s = _head_proj(att_src2, 1, OUT_DIM)
    B2d = _head_proj(att_dst2, 1, OUT_DIM)

    rxmat = jnp.zeros((16, IN_DIM), jnp.float32)
    rxmat = rxmat.at[jnp.repeat(jnp.arange(HEADS), HID),
                     jnp.arange(IN_DIM)].set(1.0)
    cxmat = jnp.zeros((16, OUT_DIM), jnp.float32).at[0, :].set(1.0)

    h1, sp1, dp1 = _prep1(x, W1, Bs1, Bd1)
    den1, ea1 = _edge_phase_a(sp1, dp1, src_rows, dst_rows)
    P = _edge_phase_b(h1, ea1, src_rows, dst_rows, IN_DIM)
    h2, sp2, dp2 = _comb1(P[0], P[1], den1[0], den1[1], h1, W2, B2s, B2d,
                          b1[None, :], rxmat)
    den2, ea2 = _edge_phase_a(sp2, dp2, src_rows, dst_rows)
    Q = _edge_phase_b(h2, ea2, src_rows, dst_rows, OUT_DIM)
    return _comb2(Q[0], Q[1], den2[0], den2[1], h2, b2[None, :], cxmat)


# speedup vs baseline: 42.4248x; 42.4248x over previous
import functools

import jax
import jax.numpy as jnp
from jax import lax
from jax.experimental import pallas as pl
from jax.experimental.pallas import tpu as pltpu
from jax.experimental.pallas import tpu_sc as plsc

N = 10000
E = 320000
IN_DIM = 128
HID = 16
HEADS = 8
OUT_DIM = 16

NC, NS = 2, 16
NW = NC * NS
EB = 128
CPW = 80
EP = NW * CPW * EB
ROWS = EP // EB
VROWS = E // EB
NP = 10240
RPT = NP // NS
BLK = 1024

NEG = 0.2
_DOT = dict(preferred_element_type=jnp.float32, precision=lax.Precision.HIGHEST)


def _prep1_body(x_ref, w1_ref, bs_ref, bd_ref, h_ref, sp_ref, dp_ref):
    h = jnp.dot(x_ref[...], w1_ref[...], **_DOT)
    s = jnp.dot(h, bs_ref[...], **_DOT)
    d = jnp.dot(h, bd_ref[...], **_DOT)
    z = s + d
    c = jnp.maximum(z, NEG * z)
    lane = lax.broadcasted_iota(jnp.int32, c.shape, 1)
    eneg = jnp.where(lane < HEADS, jnp.exp(-c), 0.0)
    h_ref[...] = h
    sp_ref[...] = s
    dp_ref[...] = jnp.concatenate([d, eneg], axis=1)


def _prep1(x, W1, Bs, Bd):
    return pl.pallas_call(
        _prep1_body,
        grid=(NP // BLK,),
        in_specs=[
            pl.BlockSpec((BLK, IN_DIM), lambda i: (i, 0)),
            pl.BlockSpec((IN_DIM, IN_DIM), lambda i: (0, 0)),
            pl.BlockSpec((IN_DIM, 16), lambda i: (0, 0)),
            pl.BlockSpec((IN_DIM, 16), lambda i: (0, 0)),
        ],
        out_specs=[
            pl.BlockSpec((BLK, IN_DIM), lambda i: (i, 0)),
            pl.BlockSpec((BLK, 16), lambda i: (i, 0)),
            pl.BlockSpec((BLK, 32), lambda i: (i, 0)),
        ],
        out_shape=[
            jax.ShapeDtypeStruct((NP, IN_DIM), jnp.float32),
            jax.ShapeDtypeStruct((NP, 16), jnp.float32),
            jax.ShapeDtypeStruct((NP, 32), jnp.float32),
        ],
    )(x, W1, Bs, Bd)


def _edge_phase_a(spack, dpack, src_rows, dst_rows):
    mesh = plsc.VectorSubcoreMesh(
        core_axis_name="c", subcore_axis_name="s", num_cores=NC, num_subcores=NS
    )

    @functools.partial(
        pl.kernel,
        out_type=[
            jax.ShapeDtypeStruct((NC, NP, 16), jnp.float32),
            jax.ShapeDtypeStruct((EP, 16), jnp.float32),
        ],
        mesh=mesh,
        scratch_types=[
            pltpu.VMEM((CPW, EB), jnp.int32),
            pltpu.VMEM((CPW, EB), jnp.int32),
            pltpu.VMEM((EB, 16), jnp.float32),
            pltpu.VMEM((EB, 32), jnp.float32),
            pltpu.VMEM((EB, 16), jnp.float32),
            pltpu.VMEM_SHARED((NP, 16), jnp.float32),
        ],
        compiler_params=pltpu.CompilerParams(use_tc_tiling_on_sc=False),
    )
    def k(sp_hbm, dp_hbm, src_hbm, dst_hbm, den_hbm, ea_hbm,
          ids_s, ids_d, sbuf, dbuf, ebuf, den_sh):
        ci = lax.axis_index("c")
        si = lax.axis_index("s")
        w = si * NC + ci
        base_row = w * CPW
        pltpu.sync_copy(src_hbm.at[pl.ds(base_row, CPW)], ids_s)
        pltpu.sync_copy(dst_hbm.at[pl.ds(base_row, CPW)], ids_d)

        @pl.loop(0, EB)
        def _(i):
            ebuf[i, :] = jnp.zeros((16,), jnp.float32)

        @pl.loop(0, RPT // EB)
        def _(r):
            pltpu.sync_copy(ebuf, den_sh.at[pl.ds(si * RPT + r * EB, EB)])

        plsc.subcore_barrier()

        @pl.loop(0, CPW)
        def _(j):
            @pl.when(base_row + j < VROWS)
            def _():
                pltpu.sync_copy(sp_hbm.at[ids_s.at[j]], sbuf)
                pltpu.sync_copy(dp_hbm.at[ids_d.at[j]], dbuf)

                @pl.loop(0, EB)
                def _(e):
                    sv = sbuf[e, :]
                    d1 = dbuf[e, pl.ds(0, 16)]
                    d2 = dbuf[e, pl.ds(16, 16)]
                    z = sv + d1
                    lz = jnp.maximum(z, NEG * z)
                    ebuf[e, :] = jnp.exp(lz) * d2

                pltpu.sync_copy(ebuf, den_sh.at[ids_d.at[j]], add=True)
                pltpu.sync_copy(ebuf, ea_hbm.at[pl.ds((base_row + j) * EB, EB)])

        plsc.subcore_barrier()
        pltpu.sync_copy(den_sh.at[pl.ds(si * RPT, RPT)],
                        den_hbm.at[ci].at[pl.ds(si * RPT, RPT)])

    return k(spack, dpack, src_rows, dst_rows)


def _edge_phase_b(h, ealpha, src_rows, dst_rows, D):
    nvec = D // 16
    mesh = plsc.VectorSubcoreMesh(
        core_axis_name="c", subcore_axis_name="s", num_cores=NC, num_subcores=NS
    )

    @functools.partial(
        pl.kernel,
        out_type=jax.ShapeDtypeStruct((NC, NP, D), jnp.float32),
        mesh=mesh,
        scratch_types=[
            pltpu.VMEM((CPW, EB), jnp.int32),
            pltpu.VMEM((CPW, EB), jnp.int32),
            pltpu.VMEM((EB, D), jnp.float32),
            pltpu.VMEM((EB, 16), jnp.float32),
            pltpu.VMEM_SHARED((NP, D), jnp.float32),
        ],
        compiler_params=pltpu.CompilerParams(use_tc_tiling_on_sc=False),
    )
    def k(h_hbm, ea_hbm, src_hbm, dst_hbm, out_hbm,
          ids_s, ids_d, hbuf, ebuf, out_sh):
        ci = lax.axis_index("c")
        si = lax.axis_index("s")
        w = si * NC + ci
        base_row = w * CPW
        pltpu.sync_copy(src_hbm.at[pl.ds(base_row, CPW)], ids_s)
        pltpu.sync_copy(dst_hbm.at[pl.ds(base_row, CPW)], ids_d)

        @pl.loop(0, EB)
        def _(i):
            for v in range(nvec):
                hbuf[i, pl.ds(v * 16, 16)] = jnp.zeros((16,), jnp.float32)

        @pl.loop(0, RPT // EB)
        def _(r):
            pltpu.sync_copy(hbuf, out_sh.at[pl.ds(si * RPT + r * EB, EB)])

        plsc.subcore_barrier()

        @pl.loop(0, CPW)
        def _(j):
            @pl.when(base_row + j < VROWS)
            def _():
                pltpu.sync_copy(h_hbm.at[ids_s.at[j]], hbuf)
                pltpu.sync_copy(ea_hbm.at[pl.ds((base_row + j) * EB, EB)], ebuf)

                @pl.loop(0, EB)
                def _(e):
                    ev = ebuf[e, :]
                    for v in range(nvec):
                        hv = hbuf[e, pl.ds(v * 16, 16)]
                        hbuf[e, pl.ds(v * 16, 16)] = hv * ev[v]

                pltpu.sync_copy(hbuf, out_sh.at[ids_d.at[j]], add=True)

        plsc.subcore_barrier()
        pltpu.sync_copy(out_sh.at[pl.ds(si * RPT, RPT)],
                        out_hbm.at[ci].at[pl.ds(si * RPT, RPT)])

    return k(h, ealpha, src_rows, dst_rows)


def _comb1_body(p0_ref, p1_ref, d0_ref, d1_ref, h1_ref, w2_ref, b2s_ref,
                b2d_ref, b1_ref, rx_ref, h2_ref, sp2_ref, dp2_ref):
    den = d0_ref[...] + d1_ref[...] + 1.0
    rcp = 1.0 / (den + 1e-16)
    rexp = jnp.dot(rcp, rx_ref[...], **_DOT)
    g = (p0_ref[...] + p1_ref[...] + h1_ref[...]) * rexp + b1_ref[...]
    g = jnp.where(g > 0, g, jnp.exp(g) - 1.0)
    h2 = jnp.dot(g, w2_ref[...], **_DOT)
    s2 = jnp.dot(h2, b2s_ref[...], **_DOT)
    d2 = jnp.dot(h2, b2d_ref[...], **_DOT)
    z = s2 + d2
    c2 = jnp.maximum(z, NEG * z)
    lane = lax.broadcasted_iota(jnp.int32, c2.shape, 1)
    eneg = jnp.where(lane < 1, jnp.exp(-c2), 0.0)
    h2_ref[...] = h2
    sp2_ref[...] = s2
    dp2_ref[...] = jnp.concatenate([d2, eneg], axis=1)


def _comb1(p0, p1, d0, d1, h1, W2, B2s, B2d, b1, rxmat):
    return pl.pallas_call(
        _comb1_body,
        grid=(NP // BLK,),
        in_specs=[
            pl.BlockSpec((BLK, IN_DIM), lambda i: (i, 0)),
            pl.BlockSpec((BLK, IN_DIM), lambda i: (i, 0)),
            pl.BlockSpec((BLK, 16), lambda i: (i, 0)),
            pl.BlockSpec((BLK, 16), lambda i: (i, 0)),
            pl.BlockSpec((BLK, IN_DIM), lambda i: (i, 0)),
            pl.BlockSpec((IN_DIM, OUT_DIM), lambda i: (0, 0)),
            pl.BlockSpec((OUT_DIM, 16), lambda i: (0, 0)),
            pl.BlockSpec((OUT_DIM, 16), lambda i: (0, 0)),
            pl.BlockSpec((1, IN_DIM), lambda i: (0, 0)),
            pl.BlockSpec((16, IN_DIM), lambda i: (0, 0)),
        ],
        out_specs=[
            pl.BlockSpec((BLK, OUT_DIM), lambda i: (i, 0)),
            pl.BlockSpec((BLK, 16), lambda i: (i, 0)),
            pl.BlockSpec((BLK, 32), lambda i: (i, 0)),
        ],
        out_shape=[
            jax.ShapeDtypeStruct((NP, OUT_DIM), jnp.float32),
            jax.ShapeDtypeStruct((NP, 16), jnp.float32),
            jax.ShapeDtypeStruct((NP, 32), jnp.float32),
        ],
    )(p0, p1, d0, d1, h1, W2, B2s, B2d, b1, rxmat)


def _comb2_body(q0_ref, q1_ref, dd0_ref, dd1_ref, h2_ref, b2_ref, cx_ref,
                out_ref):
    den = dd0_ref[...] + dd1_ref[...] + 1.0
    rcp = 1.0 / (den + 1e-16)
    r0 = jnp.dot(rcp, cx_ref[...], **_DOT)
    out_ref[...] = (q0_ref[...] + q1_ref[...] + h2_ref[...]) * r0 + b2_ref[...]


def _comb2(q0, q1, dd0, dd1, h2, b2, cxmat):
    return pl.pallas_call(
        _comb2_body,
        grid=(NP // BLK,),
        in_specs=[
            pl.BlockSpec((BLK, OUT_DIM), lambda i: (i, 0)),
            pl.BlockSpec((BLK, OUT_DIM), lambda i: (i, 0)),
            pl.BlockSpec((BLK, 16), lambda i: (i, 0)),
            pl.BlockSpec((BLK, 16), lambda i: (i, 0)),
            pl.BlockSpec((BLK, OUT_DIM), lambda i: (i, 0)),
            pl.BlockSpec((1, OUT_DIM), lambda i: (0, 0)),
            pl.BlockSpec((16, OUT_DIM), lambda i: (0, 0)),
        ],
        out_specs=pl.BlockSpec((BLK, OUT_DIM), lambda i: (i, 0)),
        out_shape=jax.ShapeDtypeStruct((NP, OUT_DIM), jnp.float32),
    )(q0, q1, dd0, dd1, h2, b2, cxmat)


def _head_proj(att, heads, hid):
    m = jnp.zeros((heads * hid, 16), jnp.float32)
    rows = jnp.arange(heads * hid)
    cols = jnp.repeat(jnp.arange(heads), hid)
    return m.at[rows, cols].set(att.reshape(-1))


def kernel(x, edge_index, W1, att_src1, att_dst1, b1, W2, att_src2, att_dst2,
           b2):
    pad_e = jnp.zeros((EP - E,), jnp.int32)
    src_rows = jnp.concatenate([edge_index[0], pad_e]).reshape(ROWS, EB)
    dst_rows = jnp.concatenate([edge_index[1], pad_e]).reshape(ROWS, EB)
    x_pad = jnp.concatenate([x, jnp.zeros((NP - N, IN_DIM), jnp.float32)])

    Bs1 = _head_proj(att_src1, HEADS, HID)
    Bd1 = _head_proj(att_dst1, HEADS, HID)
    B2s = _head_proj(att_src2, 1, OUT_DIM)
    B2d = _head_proj(att_dst2, 1, OUT_DIM)

    rxmat = jnp.zeros((16, IN_DIM), jnp.float32)
    rxmat = rxmat.at[jnp.repeat(jnp.arange(HEADS), HID),
                     jnp.arange(IN_DIM)].set(1.0)
    cxmat = jnp.zeros((16, OUT_DIM), jnp.float32).at[0, :].set(1.0)

    h1, sp1, dp1 = _prep1(x_pad, W1, Bs1, Bd1)
    den1, ea1 = _edge_phase_a(sp1, dp1, src_rows, dst_rows)
    P = _edge_phase_b(h1, ea1, src_rows, dst_rows, IN_DIM)
    h2, sp2, dp2 = _comb1(P[0], P[1], den1[0], den1[1], h1, W2, B2s, B2d,
                          b1[None, :], rxmat)
    den2, ea2 = _edge_phase_a(sp2, dp2, src_rows, dst_rows)
    Q = _edge_phase_b(h2, ea2, src_rows, dst_rows, OUT_DIM)
    out = _comb2(Q[0], Q[1], den2[0], den2[1], h2, b2[None, :], cxmat)
    return out[:N]


# speedup vs baseline: 59.9151x; 1.4123x over previous
import functools

import jax
import jax.numpy as jnp
from jax import lax
from jax.experimental import pallas as pl
from jax.experimental.pallas import tpu as pltpu
from jax.experimental.pallas import tpu_sc as plsc

N = 10000
E = 320000
IN_DIM = 128
HID = 16
HEADS = 8
OUT_DIM = 16

NC, NS = 2, 16
NW = NC * NS
EB = 128
CPW = 80
EP = NW * CPW * EB
ROWS = EP // EB
VROWS = E // EB
NP = 10240
RPT = NP // NS
BLK = 1024

NEG = 0.2
_DOT = dict(preferred_element_type=jnp.float32, precision=lax.Precision.HIGHEST)


def _prep1_body(x_ref, w1_ref, bs_ref, bd_ref, h_ref, sp_ref, dp_ref):
    h = jnp.dot(x_ref[...], w1_ref[...], **_DOT)
    s = jnp.dot(h, bs_ref[...], **_DOT)
    d = jnp.dot(h, bd_ref[...], **_DOT)
    z = s + d
    c = jnp.maximum(z, NEG * z)
    lane = lax.broadcasted_iota(jnp.int32, c.shape, 1)
    eneg = jnp.where(lane < HEADS, jnp.exp(-c), 0.0)
    h_ref[...] = h
    sp_ref[...] = s
    dp_ref[...] = jnp.concatenate([d, eneg], axis=1)


def _prep1(x, W1, Bs, Bd):
    return pl.pallas_call(
        _prep1_body,
        grid=(NP // BLK,),
        in_specs=[
            pl.BlockSpec((BLK, IN_DIM), lambda i: (i, 0)),
            pl.BlockSpec((IN_DIM, IN_DIM), lambda i: (0, 0)),
            pl.BlockSpec((IN_DIM, 16), lambda i: (0, 0)),
            pl.BlockSpec((IN_DIM, 16), lambda i: (0, 0)),
        ],
        out_specs=[
            pl.BlockSpec((BLK, IN_DIM), lambda i: (i, 0)),
            pl.BlockSpec((BLK, 16), lambda i: (i, 0)),
            pl.BlockSpec((BLK, 32), lambda i: (i, 0)),
        ],
        out_shape=[
            jax.ShapeDtypeStruct((NP, IN_DIM), jnp.float32),
            jax.ShapeDtypeStruct((NP, 16), jnp.float32),
            jax.ShapeDtypeStruct((NP, 32), jnp.float32),
        ],
    )(x, W1, Bs, Bd)


def _edge_phase_a(spack, dpack, src_rows, dst_rows):
    mesh = plsc.VectorSubcoreMesh(
        core_axis_name="c", subcore_axis_name="s", num_cores=NC, num_subcores=NS
    )

    @functools.partial(
        pl.kernel,
        out_type=[
            jax.ShapeDtypeStruct((NC, NP, 16), jnp.float32),
            jax.ShapeDtypeStruct((EP, 16), jnp.float32),
        ],
        mesh=mesh,
        scratch_types=[
            pltpu.VMEM((CPW, EB), jnp.int32),
            pltpu.VMEM((CPW, EB), jnp.int32),
            pltpu.VMEM((2, EB, 16), jnp.float32),
            pltpu.VMEM((2, EB, 32), jnp.float32),
            pltpu.VMEM((2, EB, 16), jnp.float32),
            pltpu.VMEM_SHARED((NP, 16), jnp.float32),
            pltpu.SemaphoreType.DMA,
            pltpu.SemaphoreType.DMA,
            pltpu.SemaphoreType.DMA,
            pltpu.SemaphoreType.DMA,
            pltpu.SemaphoreType.DMA,
            pltpu.SemaphoreType.DMA,
            pltpu.SemaphoreType.DMA,
            pltpu.SemaphoreType.DMA,
        ],
        compiler_params=pltpu.CompilerParams(use_tc_tiling_on_sc=False),
    )
    def k(sp_hbm, dp_hbm, src_hbm, dst_hbm, den_hbm, ea_hbm,
          ids_s, ids_d, sbuf, dbuf, ebuf, den_sh,
          ga0s, gb0s, ga1s, gb1s, sa0s, wa0s, sa1s, wa1s):
        ci = lax.axis_index("c")
        si = lax.axis_index("s")
        w = si * NC + ci
        base_row = w * CPW
        nvalid = jnp.clip(VROWS - base_row, 0, CPW)
        pltpu.sync_copy(src_hbm.at[pl.ds(base_row, CPW)], ids_s)
        pltpu.sync_copy(dst_hbm.at[pl.ds(base_row, CPW)], ids_d)

        @pl.loop(0, EB)
        def _(i):
            ebuf[0, i, :] = jnp.zeros((16,), jnp.float32)

        @pl.loop(0, RPT // EB)
        def _(r):
            pltpu.sync_copy(ebuf.at[0], den_sh.at[pl.ds(si * RPT + r * EB, EB)])

        plsc.subcore_barrier()

        def compute(b):
            @pl.loop(0, EB)
            def _(e):
                sv = sbuf[b, e, :]
                d1 = dbuf[b, e, pl.ds(0, 16)]
                d2 = dbuf[b, e, pl.ds(16, 16)]
                z = sv + d1
                lz = jnp.maximum(z, NEG * z)
                ebuf[b, e, :] = jnp.exp(lz) * d2

        @pl.loop(0, CPW // 2)
        def _(jj):
            j0 = 2 * jj
            j1 = j0 + 1

            @pl.when(j0 < nvalid)
            def _():
                ga0 = pltpu.async_copy(sp_hbm.at[ids_s.at[j0]], sbuf.at[0],
                                       ga0s)
                gb0 = pltpu.async_copy(dp_hbm.at[ids_d.at[j0]], dbuf.at[0],
                                       gb0s)
                ga1 = pltpu.async_copy(sp_hbm.at[ids_s.at[j1]], sbuf.at[1],
                                       ga1s)
                gb1 = pltpu.async_copy(dp_hbm.at[ids_d.at[j1]], dbuf.at[1],
                                       gb1s)
                ga0.wait()
                gb0.wait()
                compute(0)
                sa0 = pltpu.async_copy(ebuf.at[0], den_sh.at[ids_d.at[j0]],
                                       sa0s, add=True)
                wa0 = pltpu.async_copy(
                    ebuf.at[0], ea_hbm.at[pl.ds((base_row + j0) * EB, EB)],
                    wa0s)
                ga1.wait()
                gb1.wait()
                compute(1)
                sa1 = pltpu.async_copy(ebuf.at[1], den_sh.at[ids_d.at[j1]],
                                       sa1s, add=True)
                wa1 = pltpu.async_copy(
                    ebuf.at[1], ea_hbm.at[pl.ds((base_row + j1) * EB, EB)],
                    wa1s)
                sa0.wait()
                wa0.wait()
                sa1.wait()
                wa1.wait()

        plsc.subcore_barrier()
        pltpu.sync_copy(den_sh.at[pl.ds(si * RPT, RPT)],
                        den_hbm.at[ci].at[pl.ds(si * RPT, RPT)])

    return k(spack, dpack, src_rows, dst_rows)


def _edge_phase_b(h, ealpha, src_rows, dst_rows, D):
    nvec = D // 16
    mesh = plsc.VectorSubcoreMesh(
        core_axis_name="c", subcore_axis_name="s", num_cores=NC, num_subcores=NS
    )

    @functools.partial(
        pl.kernel,
        out_type=jax.ShapeDtypeStruct((NC, NP, D), jnp.float32),
        mesh=mesh,
        scratch_types=[
            pltpu.VMEM((CPW // 2, EB), jnp.int32),
            pltpu.VMEM((CPW // 2, EB), jnp.int32),
            pltpu.VMEM((2, EB, D), jnp.float32),
            pltpu.VMEM((2, EB, 16), jnp.float32),
            pltpu.VMEM_SHARED((NP, D), jnp.float32),
            pltpu.SemaphoreType.DMA,
            pltpu.SemaphoreType.DMA,
            pltpu.SemaphoreType.DMA,
            pltpu.SemaphoreType.DMA,
            pltpu.SemaphoreType.DMA,
            pltpu.SemaphoreType.DMA,
        ],
        compiler_params=pltpu.CompilerParams(use_tc_tiling_on_sc=False),
    )
    def k(h_hbm, ea_hbm, src_hbm, dst_hbm, out_hbm,
          ids_s, ids_d, hbuf, ebuf, out_sh,
          ga0s, gb0s, ga1s, gb1s, sa0s, sa1s):
        ci = lax.axis_index("c")
        si = lax.axis_index("s")
        w = si * NC + ci
        base_row = w * CPW
        nvalid = jnp.clip(VROWS - base_row, 0, CPW)

        @pl.loop(0, EB)
        def _(i):
            for v in range(nvec):
                hbuf[0, i, pl.ds(v * 16, 16)] = jnp.zeros((16,), jnp.float32)

        @pl.loop(0, RPT // EB)
        def _(r):
            pltpu.sync_copy(hbuf.at[0], out_sh.at[pl.ds(si * RPT + r * EB, EB)])

        plsc.subcore_barrier()

        def compute(b):
            @pl.loop(0, EB)
            def _(e):
                ev = ebuf[b, e, :]
                for v in range(nvec):
                    hv = hbuf[b, e, pl.ds(v * 16, 16)]
                    hbuf[b, e, pl.ds(v * 16, 16)] = hv * ev[v]

        @pl.loop(0, 2)
        def _(half):
            hbase = base_row + half * (CPW // 2)

            @pl.when(half * (CPW // 2) < nvalid)
            def _():
                pltpu.sync_copy(src_hbm.at[pl.ds(hbase, CPW // 2)], ids_s)
                pltpu.sync_copy(dst_hbm.at[pl.ds(hbase, CPW // 2)], ids_d)

                @pl.loop(0, CPW // 4)
                def _(jj):
                    j0 = 2 * jj
                    j1 = j0 + 1

                    @pl.when(half * (CPW // 2) + j0 < nvalid)
                    def _():
                        ga0 = pltpu.async_copy(h_hbm.at[ids_s.at[j0]],
                                               hbuf.at[0], ga0s)
                        gb0 = pltpu.async_copy(
                            ea_hbm.at[pl.ds((hbase + j0) * EB, EB)],
                            ebuf.at[0], gb0s)
                        ga1 = pltpu.async_copy(h_hbm.at[ids_s.at[j1]],
                                               hbuf.at[1], ga1s)
                        gb1 = pltpu.async_copy(
                            ea_hbm.at[pl.ds((hbase + j1) * EB, EB)],
                            ebuf.at[1], gb1s)
                        ga0.wait()
                        gb0.wait()
                        compute(0)
                        sa0 = pltpu.async_copy(hbuf.at[0],
                                               out_sh.at[ids_d.at[j0]], sa0s,
                                               add=True)
                        ga1.wait()
                        gb1.wait()
                        compute(1)
                        sa1 = pltpu.async_copy(hbuf.at[1],
                                               out_sh.at[ids_d.at[j1]], sa1s,
                                               add=True)
                        sa0.wait()
                        sa1.wait()

        plsc.subcore_barrier()
        pltpu.sync_copy(out_sh.at[pl.ds(si * RPT, RPT)],
                        out_hbm.at[ci].at[pl.ds(si * RPT, RPT)])

    return k(h, ealpha, src_rows, dst_rows)


def _comb1_body(p0_ref, p1_ref, d0_ref, d1_ref, h1_ref, w2_ref, b2s_ref,
                b2d_ref, b1_ref, rx_ref, h2_ref, sp2_ref, dp2_ref):
    den = d0_ref[...] + d1_ref[...] + 1.0
    rcp = 1.0 / (den + 1e-16)
    rexp = jnp.dot(rcp, rx_ref[...], **_DOT)
    g = (p0_ref[...] + p1_ref[...] + h1_ref[...]) * rexp + b1_ref[...]
    g = jnp.where(g > 0, g, jnp.exp(g) - 1.0)
    h2 = jnp.dot(g, w2_ref[...], **_DOT)
    s2 = jnp.dot(h2, b2s_ref[...], **_DOT)
    d2 = jnp.dot(h2, b2d_ref[...], **_DOT)
    z = s2 + d2
    c2 = jnp.maximum(z, NEG * z)
    lane = lax.broadcasted_iota(jnp.int32, c2.shape, 1)
    eneg = jnp.where(lane < 1, jnp.exp(-c2), 0.0)
    h2_ref[...] = h2
    sp2_ref[...] = s2
    dp2_ref[...] = jnp.concatenate([d2, eneg], axis=1)


def _comb1(p0, p1, d0, d1, h1, W2, B2s, B2d, b1, rxmat):
    return pl.pallas_call(
        _comb1_body,
        grid=(NP // BLK,),
        in_specs=[
            pl.BlockSpec((BLK, IN_DIM), lambda i: (i, 0)),
            pl.BlockSpec((BLK, IN_DIM), lambda i: (i, 0)),
            pl.BlockSpec((BLK, 16), lambda i: (i, 0)),
            pl.BlockSpec((BLK, 16), lambda i: (i, 0)),
            pl.BlockSpec((BLK, IN_DIM), lambda i: (i, 0)),
            pl.BlockSpec((IN_DIM, OUT_DIM), lambda i: (0, 0)),
            pl.BlockSpec((OUT_DIM, 16), lambda i: (0, 0)),
            pl.BlockSpec((OUT_DIM, 16), lambda i: (0, 0)),
            pl.BlockSpec((1, IN_DIM), lambda i: (0, 0)),
            pl.BlockSpec((16, IN_DIM), lambda i: (0, 0)),
        ],
        out_specs=[
            pl.BlockSpec((BLK, OUT_DIM), lambda i: (i, 0)),
            pl.BlockSpec((BLK, 16), lambda i: (i, 0)),
            pl.BlockSpec((BLK, 32), lambda i: (i, 0)),
        ],
        out_shape=[
            jax.ShapeDtypeStruct((NP, OUT_DIM), jnp.float32),
            jax.ShapeDtypeStruct((NP, 16), jnp.float32),
            jax.ShapeDtypeStruct((NP, 32), jnp.float32),
        ],
    )(p0, p1, d0, d1, h1, W2, B2s, B2d, b1, rxmat)


def _comb2_body(q0_ref, q1_ref, dd0_ref, dd1_ref, h2_ref, b2_ref, cx_ref,
                out_ref):
    den = dd0_ref[...] + dd1_ref[...] + 1.0
    rcp = 1.0 / (den + 1e-16)
    r0 = jnp.dot(rcp, cx_ref[...], **_DOT)
    out_ref[...] = (q0_ref[...] + q1_ref[...] + h2_ref[...]) * r0 + b2_ref[...]


def _comb2(q0, q1, dd0, dd1, h2, b2, cxmat):
    return pl.pallas_call(
        _comb2_body,
        grid=(NP // BLK,),
        in_specs=[
            pl.BlockSpec((BLK, OUT_DIM), lambda i: (i, 0)),
            pl.BlockSpec((BLK, OUT_DIM), lambda i: (i, 0)),
            pl.BlockSpec((BLK, 16), lambda i: (i, 0)),
            pl.BlockSpec((BLK, 16), lambda i: (i, 0)),
            pl.BlockSpec((BLK, OUT_DIM), lambda i: (i, 0)),
            pl.BlockSpec((1, OUT_DIM), lambda i: (0, 0)),
            pl.BlockSpec((16, OUT_DIM), lambda i: (0, 0)),
        ],
        out_specs=pl.BlockSpec((BLK, OUT_DIM), lambda i: (i, 0)),
        out_shape=jax.ShapeDtypeStruct((NP, OUT_DIM), jnp.float32),
    )(q0, q1, dd0, dd1, h2, b2, cxmat)


def _head_proj(att, heads, hid):
    m = jnp.zeros((heads * hid, 16), jnp.float32)
    rows = jnp.arange(heads * hid)
    cols = jnp.repeat(jnp.arange(heads), hid)
    return m.at[rows, cols].set(att.reshape(-1))


def kernel(x, edge_index, W1, att_src1, att_dst1, b1, W2, att_src2, att_dst2,
           b2):
    pad_e = jnp.zeros((EP - E,), jnp.int32)
    src_rows = jnp.concatenate([edge_index[0], pad_e]).reshape(ROWS, EB)
    dst_rows = jnp.concatenate([edge_index[1], pad_e]).reshape(ROWS, EB)
    x_pad = jnp.concatenate([x, jnp.zeros((NP - N, IN_DIM), jnp.float32)])

    Bs1 = _head_proj(att_src1, HEADS, HID)
    Bd1 = _head_proj(att_dst1, HEADS, HID)
    B2s = _head_proj(att_src2, 1, OUT_DIM)
    B2d = _head_proj(att_dst2, 1, OUT_DIM)

    rxmat = jnp.zeros((16, IN_DIM), jnp.float32)
    rxmat = rxmat.at[jnp.repeat(jnp.arange(HEADS), HID),
                     jnp.arange(IN_DIM)].set(1.0)
    cxmat = jnp.zeros((16, OUT_DIM), jnp.float32).at[0, :].set(1.0)

    h1, sp1, dp1 = _prep1(x_pad, W1, Bs1, Bd1)
    den1, ea1 = _edge_phase_a(sp1, dp1, src_rows, dst_rows)
    P = _edge_phase_b(h1, ea1, src_rows, dst_rows, IN_DIM)
    h2, sp2, dp2 = _comb1(P[0], P[1], den1[0], den1[1], h1, W2, B2s, B2d,
                          b1[None, :], rxmat)
    den2, ea2 = _edge_phase_a(sp2, dp2, src_rows, dst_rows)
    Q = _edge_phase_b(h2, ea2, src_rows, dst_rows, OUT_DIM)
    out = _comb2(Q[0], Q[1], den2[0], den2[1], h2, b2[None, :], cxmat)
    return out[:N]
